# baseline (device time: 12968 ns/iter reference)
import jax
import jax.numpy as jnp
from jax import lax
from jax.experimental import pallas as pl
from jax.experimental.pallas import tpu as pltpu

N_DEV = 4
N_CHUNK = 8


def kernel(x):
    m_per, n = x.shape
    ch = m_per // N_CHUNK

    def body(x_hbm, out_ref, buf_ref, comm_ref, copy_sems, send_sems, recv_sems):
        my_pos = lax.axis_index("i")

        barrier_sem = pltpu.get_barrier_semaphore()
        for d in range(1, N_DEV):
            pl.semaphore_signal(
                barrier_sem, inc=1,
                device_id=((my_pos + d) % N_DEV,),
                device_id_type=pl.DeviceIdType.MESH,
            )

        def chunk_copy(k):
            return pltpu.make_async_copy(
                x_hbm.at[pl.ds(k * ch, ch), :],
                buf_ref.at[k],
                copy_sems.at[k],
            )

        for k in range(N_CHUNK):
            chunk_copy(k).start()
        acc = None
        for k in range(N_CHUNK):
            chunk_copy(k).wait()
            m = jnp.max(buf_ref[k], axis=0, keepdims=True)
            acc = m if acc is None else jnp.maximum(acc, m)
        comm_ref[0, :, :] = acc

        pl.semaphore_wait(barrier_sem, N_DEV - 1)

        rdmas = []
        for d in range(1, N_DEV):
            rdma = pltpu.make_async_remote_copy(
                src_ref=comm_ref.at[0],
                dst_ref=comm_ref.at[d],
                send_sem=send_sems.at[d - 1],
                recv_sem=recv_sems.at[d - 1],
                device_id=((my_pos + d) % N_DEV,),
                device_id_type=pl.DeviceIdType.MESH,
            )
            rdma.start()
            rdmas.append(rdma)

        for rdma in rdmas:
            rdma.wait()

        out_ref[:, :] = jnp.maximum(
            jnp.maximum(comm_ref[0, :, :], comm_ref[1, :, :]),
            jnp.maximum(comm_ref[2, :, :], comm_ref[3, :, :]),
        )

    return pl.pallas_call(
        body,
        out_shape=jax.ShapeDtypeStruct((1, n), jnp.float32),
        in_specs=[pl.BlockSpec(memory_space=pl.ANY)],
        out_specs=pl.BlockSpec(memory_space=pltpu.VMEM),
        scratch_shapes=[
            pltpu.VMEM((N_CHUNK, ch, n), jnp.float32),
            pltpu.VMEM((N_DEV, 1, n), jnp.float32),
            pltpu.SemaphoreType.DMA((N_CHUNK,)),
            pltpu.SemaphoreType.DMA((N_DEV - 1,)),
            pltpu.SemaphoreType.DMA((N_DEV - 1,)),
        ],
        compiler_params=pltpu.CompilerParams(collective_id=0),
    )(x)


# device time: 12856 ns/iter; 1.0087x vs baseline; 1.0087x over previous
import jax
import jax.numpy as jnp
from jax import lax
from jax.experimental import pallas as pl
from jax.experimental.pallas import tpu as pltpu

N_DEV = 4
N_CHUNK = 8
DUMMY_ROWS = 1536


def kernel(x):
    m_per, n = x.shape
    ch = m_per // N_CHUNK

    def body(x_hbm, out_ref, dummy_ref, buf_ref, comm_ref, res_ref,
             copy_sems, out_sem, send_sems, recv_sems):
        my_pos = lax.axis_index("i")

        barrier_sem = pltpu.get_barrier_semaphore()
        for d in range(1, N_DEV):
            pl.semaphore_signal(
                barrier_sem, inc=1,
                device_id=((my_pos + d) % N_DEV,),
                device_id_type=pl.DeviceIdType.MESH,
            )

        def chunk_copy(k):
            return pltpu.make_async_copy(
                x_hbm.at[pl.ds(k * ch, ch), :],
                buf_ref.at[k % 2],
                copy_sems.at[k % 2],
            )

        chunk_copy(0).start()
        acc = None
        for k in range(N_CHUNK):
            if k + 1 < N_CHUNK:
                chunk_copy(k + 1).start()
            chunk_copy(k).wait()
            m = jnp.max(buf_ref[k % 2], axis=0, keepdims=True)
            acc = m if acc is None else jnp.maximum(acc, m)
        comm_ref[0, :, :] = acc

        pl.semaphore_wait(barrier_sem, N_DEV - 1)

        rdmas = []
        for d in range(1, N_DEV):
            rdma = pltpu.make_async_remote_copy(
                src_ref=comm_ref.at[0],
                dst_ref=comm_ref.at[d],
                send_sem=send_sems.at[d - 1],
                recv_sem=recv_sems.at[d - 1],
                device_id=((my_pos + d) % N_DEV,),
                device_id_type=pl.DeviceIdType.MESH,
            )
            rdma.start()
            rdmas.append(rdma)

        for d, rdma in enumerate(rdmas, start=1):
            rdma.wait_recv()
            acc = jnp.maximum(acc, comm_ref[d, :, :])
        res_ref[:, :] = acc
        out_copy = pltpu.make_async_copy(res_ref, out_ref, out_sem)
        out_copy.start()
        for rdma in rdmas:
            rdma.wait_send()
        out_copy.wait()

    out, _ = pl.pallas_call(
        body,
        out_shape=(
            jax.ShapeDtypeStruct((1, n), jnp.float32),
            jax.ShapeDtypeStruct((DUMMY_ROWS, 1024), jnp.float32),
        ),
        in_specs=[pl.BlockSpec(memory_space=pltpu.MemorySpace.HBM)],
        out_specs=(
            pl.BlockSpec(memory_space=pltpu.MemorySpace.HBM),
            pl.BlockSpec(memory_space=pltpu.MemorySpace.HBM),
        ),
        scratch_shapes=[
            pltpu.VMEM((2, ch, n), jnp.float32),
            pltpu.VMEM((N_DEV, 1, n), jnp.float32),
            pltpu.VMEM((1, n), jnp.float32),
            pltpu.SemaphoreType.DMA((2,)),
            pltpu.SemaphoreType.DMA,
            pltpu.SemaphoreType.DMA((N_DEV - 1,)),
            pltpu.SemaphoreType.DMA((N_DEV - 1,)),
        ],
        compiler_params=pltpu.CompilerParams(collective_id=0),
    )(x)
    return out


# device time: 11748 ns/iter; 1.1038x vs baseline; 1.0943x over previous
import jax
import jax.numpy as jnp
from jax import lax
from jax.experimental import pallas as pl
from jax.experimental.pallas import tpu as pltpu

N_DEV = 4


def kernel(x):
    m_per, n = x.shape

    def body(x_ref, out_ref, comm_ref, send_sems, recv_sems):
        my_pos = lax.axis_index("i")

        barrier_sem = pltpu.get_barrier_semaphore()
        for d in range(1, N_DEV):
            pl.semaphore_signal(
                barrier_sem, inc=1,
                device_id=((my_pos + d) % N_DEV,),
                device_id_type=pl.DeviceIdType.MESH,
            )

        acc = jnp.max(x_ref[:, :], axis=0, keepdims=True)
        comm_ref[0, :, :] = acc

        pl.semaphore_wait(barrier_sem, N_DEV - 1)

        rdmas = {}
        for d in (2, 1, 3):
            rdma = pltpu.make_async_remote_copy(
                src_ref=comm_ref.at[0],
                dst_ref=comm_ref.at[d],
                send_sem=send_sems.at[d - 1],
                recv_sem=recv_sems.at[d - 1],
                device_id=((my_pos + d) % N_DEV,),
                device_id_type=pl.DeviceIdType.MESH,
            )
            rdma.start()
            rdmas[d] = rdma

        for d in (1, 3, 2):
            rdmas[d].wait_recv()
            acc = jnp.maximum(acc, comm_ref[d, :, :])
        out_ref[:, :] = acc
        for d in (1, 2, 3):
            rdmas[d].wait_send()

    return pl.pallas_call(
        body,
        out_shape=jax.ShapeDtypeStruct((1, n), jnp.float32),
        in_specs=[pl.BlockSpec(memory_space=pltpu.MemorySpace.VMEM)],
        out_specs=pl.BlockSpec(memory_space=pltpu.MemorySpace.VMEM),
        scratch_shapes=[
            pltpu.VMEM((N_DEV, 1, n), jnp.float32),
            pltpu.SemaphoreType.DMA((N_DEV - 1,)),
            pltpu.SemaphoreType.DMA((N_DEV - 1,)),
        ],
        compiler_params=pltpu.CompilerParams(collective_id=0),
    )(x)
